# baseline (speedup 1.0000x reference)
import functools
import jax
import jax.numpy as jnp
from jax import lax
from jax.experimental import pallas as pl
from jax.experimental.pallas import tpu as pltpu, tpu_sc as plsc

DIM = 768
B_TOT = 1024 * 200
NC = 2
NS = 16
NW = NC * NS
B_PER_W = B_TOT // NW
CHUNK = 40
NBUF = 4
N_CHUNK = B_PER_W // CHUNK
N_GROUP = N_CHUNK // NBUF

_mesh = plsc.VectorSubcoreMesh(core_axis_name="c", subcore_axis_name="s")


@functools.partial(
    pl.kernel,
    mesh=_mesh,
    out_type=jax.ShapeDtypeStruct((B_TOT, DIM), jnp.float32),
    scratch_types=[
        pltpu.VMEM((B_PER_W,), jnp.int32),
    ] + [pltpu.VMEM((CHUNK, DIM), jnp.float32)] * NBUF
      + [pltpu.SemaphoreType.DMA] * (2 * NBUF),
)
def _gather_kernel(ids_hbm, table_hbm, out_hbm, idx_v, *scratch):
    bufs = scratch[:NBUF]
    sgs = scratch[NBUF:2 * NBUF]
    sss = scratch[2 * NBUF:]
    wid = lax.axis_index("s") * NC + lax.axis_index("c")
    base = wid * B_PER_W
    pltpu.sync_copy(ids_hbm.at[pl.ds(base, B_PER_W)], idx_v)

    def g_start(c, b):
        pltpu.async_copy(
            table_hbm.at[idx_v.at[pl.ds(c * CHUNK, CHUNK)]], bufs[b], sgs[b])

    def g_wait(b):
        pltpu.make_async_copy(
            table_hbm.at[idx_v.at[pl.ds(0, CHUNK)]], bufs[b], sgs[b]).wait()

    def s_start(c, b):
        pltpu.async_copy(
            bufs[b], out_hbm.at[pl.ds(base + c * CHUNK, CHUNK)], sss[b])

    def s_wait(b):
        pltpu.make_async_copy(
            bufs[b], out_hbm.at[pl.ds(base, CHUNK)], sss[b]).wait()

    for b in range(NBUF):
        g_start(b, b)

    def body(i, carry):
        c0 = NBUF * i
        for b in range(NBUF):
            g_wait(b)
            s_start(c0 + b, b)
        for b in range(NBUF):
            s_wait(b)
            g_start(c0 + NBUF + b, b)
        return carry

    lax.fori_loop(0, N_GROUP - 1, body, 0)

    c0 = NBUF * (N_GROUP - 1)
    for b in range(NBUF):
        g_wait(b)
        s_start(c0 + b, b)
    for b in range(NBUF):
        s_wait(b)


def kernel(input_ids, table):
    ids = input_ids.reshape(-1).astype(jnp.int32)
    out = _gather_kernel(ids, table)
    return out.reshape(input_ids.shape[0], input_ids.shape[1], DIM)
